# Initial kernel scaffold; baseline (speedup 1.0000x reference)
import jax
import jax.numpy as jnp
from jax.experimental import pallas as pl


def kernel(x, edge_index, edge_attr, eW1, eb1, eg1, ebt1, eW2, eb2, eg2, ebt2, eW3, eb3, nW1, nb1, ng1, nbt1, nW2, nb2, ng2, nbt2, nW3, nb3):
    raise NotImplementedError("write your pallas kernel here")


# speedup vs baseline: 2.0289x; 2.0289x over previous
import functools

import jax
import jax.numpy as jnp
from jax import lax
from jax.experimental import pallas as pl
from jax.experimental.pallas import tpu as pltpu
from jax.experimental.pallas import tpu_sc as plsc

NC = 2
NS = 16
NW = NC * NS
EPS = 1e-5


def _silu(h):
    return h * (1.0 / (1.0 + jnp.exp(-h)))


def _bn_affine(stats, g, bt, count):
    m = stats[0:1, :] / count
    v = stats[1:2, :] / count - m * m
    a = g * lax.rsqrt(v + EPS)
    c = bt - m * a
    return a, c


def _prep_body(x_ref, wa_ref, wb_ref, xa_ref, xb_ref):
    x = x_ref[...]
    xa_ref[...] = jnp.dot(x, wa_ref[...], preferred_element_type=jnp.float32)
    xb_ref[...] = jnp.dot(x, wb_ref[...], preferred_element_type=jnp.float32)


def _prep(x, wa, wb):
    N, D = x.shape
    return pl.pallas_call(
        _prep_body,
        out_shape=(jax.ShapeDtypeStruct((N, D), jnp.float32),
                   jax.ShapeDtypeStruct((N, D), jnp.float32)),
    )(x, wa, wb)


def _sc_gather(xa, xb, dst, src, K=80):
    N, D = xa.shape
    E = dst.shape[0]
    epw = E // NW
    iters = epw // K
    nv = D // 16
    mesh = plsc.VectorSubcoreMesh(core_axis_name="c", subcore_axis_name="s")

    @functools.partial(
        pl.kernel,
        out_type=jax.ShapeDtypeStruct((E, D), jnp.float32),
        mesh=mesh,
        scratch_types=[
            pltpu.VMEM((K,), jnp.int32),
            pltpu.VMEM((K,), jnp.int32),
            pltpu.VMEM((K, D), jnp.float32),
            pltpu.VMEM((K, D), jnp.float32),
            pltpu.SemaphoreType.DMA,
            pltpu.SemaphoreType.DMA,
        ],
    )
    def gather_k(xa_hbm, xb_hbm, dst_hbm, src_hbm, out_hbm,
                 dstv, srcv, av, bv, sema, semb):
        wid = lax.axis_index("s") * NC + lax.axis_index("c")
        base0 = wid * epw

        def body(i, carry):
            base = base0 + i * K
            pltpu.sync_copy(dst_hbm.at[pl.ds(base, K)], dstv)
            pltpu.sync_copy(src_hbm.at[pl.ds(base, K)], srcv)
            ca = pltpu.async_copy(xa_hbm.at[dstv], av, sema)
            cb = pltpu.async_copy(xb_hbm.at[srcv], bv, semb)
            ca.wait()
            cb.wait()

            def row(r, c2):
                for j in range(nv):
                    sl = pl.ds(16 * j, 16)
                    av[r, sl] = av[r, sl] + bv[r, sl]
                return c2

            lax.fori_loop(0, K, row, 0)
            pltpu.sync_copy(av, out_hbm.at[pl.ds(base, K)])
            return carry

        lax.fori_loop(0, iters, body, 0)

    return gather_k(xa, xb, dst, src)


def _estats_body(h1p_ref, ea_ref, w1c_ref, b1_ref, out_ref):
    i = pl.program_id(0)
    h1 = (h1p_ref[...]
          + jnp.dot(ea_ref[...], w1c_ref[...],
                    preferred_element_type=jnp.float32)
          + b1_ref[...])
    s = jnp.sum(h1, axis=0)
    q = jnp.sum(h1 * h1, axis=0)
    blk = jnp.concatenate(
        [s[None], q[None], jnp.zeros((6, h1.shape[1]), jnp.float32)], axis=0)

    @pl.when(i == 0)
    def _():
        out_ref[...] = blk

    @pl.when(i > 0)
    def _():
        out_ref[...] += blk


def _estats(h1p, ea, w1c, b1, R):
    E, D = h1p.shape
    DE = ea.shape[1]
    grid = E // R
    return pl.pallas_call(
        _estats_body,
        grid=(grid,),
        in_specs=[
            pl.BlockSpec((R, D), lambda i: (i, 0)),
            pl.BlockSpec((R, DE), lambda i: (i, 0)),
            pl.BlockSpec((DE, D), lambda i: (0, 0)),
            pl.BlockSpec((1, D), lambda i: (0, 0)),
        ],
        out_specs=pl.BlockSpec((8, D), lambda i: (0, 0)),
        out_shape=jax.ShapeDtypeStruct((8, D), jnp.float32),
    )(h1p, ea, w1c, b1)


def _emain_body(h1p_ref, ea_ref, w1c_ref, b1_ref, st1_ref, g1_ref, bt1_ref,
                w2_ref, b2_ref, cnt_ref, h2_ref, st2_ref):
    i = pl.program_id(0)
    h1 = (h1p_ref[...]
          + jnp.dot(ea_ref[...], w1c_ref[...],
                    preferred_element_type=jnp.float32)
          + b1_ref[...])
    a1, c1 = _bn_affine(st1_ref[...], g1_ref[...], bt1_ref[...],
                        cnt_ref[0, 0])
    h1n = _silu(a1 * h1 + c1)
    h2 = jnp.dot(h1n, w2_ref[...], preferred_element_type=jnp.float32) \
        + b2_ref[...]
    h2_ref[...] = h2
    s = jnp.sum(h2, axis=0)
    q = jnp.sum(h2 * h2, axis=0)
    blk = jnp.concatenate(
        [s[None], q[None], jnp.zeros((6, h2.shape[1]), jnp.float32)], axis=0)

    @pl.when(i == 0)
    def _():
        st2_ref[...] = blk

    @pl.when(i > 0)
    def _():
        st2_ref[...] += blk


def _emain(h1p, ea, w1c, b1, st1, g1, bt1, w2, b2, cnt, R):
    E, D = h1p.shape
    DE = ea.shape[1]
    H = w2.shape[1]
    grid = E // R
    return pl.pallas_call(
        _emain_body,
        grid=(grid,),
        in_specs=[
            pl.BlockSpec((R, D), lambda i: (i, 0)),
            pl.BlockSpec((R, DE), lambda i: (i, 0)),
            pl.BlockSpec((DE, D), lambda i: (0, 0)),
            pl.BlockSpec((1, D), lambda i: (0, 0)),
            pl.BlockSpec((8, D), lambda i: (0, 0)),
            pl.BlockSpec((1, D), lambda i: (0, 0)),
            pl.BlockSpec((1, D), lambda i: (0, 0)),
            pl.BlockSpec((D, H), lambda i: (0, 0)),
            pl.BlockSpec((1, H), lambda i: (0, 0)),
            pl.BlockSpec((1, 1), lambda i: (0, 0), memory_space=pltpu.SMEM),
        ],
        out_specs=(pl.BlockSpec((R, H), lambda i: (i, 0)),
                   pl.BlockSpec((8, H), lambda i: (0, 0))),
        out_shape=(jax.ShapeDtypeStruct((E, H), jnp.float32),
                   jax.ShapeDtypeStruct((8, H), jnp.float32)),
    )(h1p, ea, w1c, b1, st1, g1, bt1, w2, b2, cnt)


def _efinal_body(h2_ref, st2_ref, g2_ref, bt2_ref, w3_ref, b3_ref, cnt_ref,
                 out_ref):
    a2, c2 = _bn_affine(st2_ref[...], g2_ref[...], bt2_ref[...],
                        cnt_ref[0, 0])
    h2n = _silu(a2 * h2_ref[...] + c2)
    out_ref[...] = jnp.dot(h2n, w3_ref[...],
                           preferred_element_type=jnp.float32) + b3_ref[...]


def _efinal(h2, st2, g2, bt2, w3aug, b3aug, cnt, R):
    E, H = h2.shape
    W = w3aug.shape[1]
    grid = E // R
    return pl.pallas_call(
        _efinal_body,
        grid=(grid,),
        in_specs=[
            pl.BlockSpec((R, H), lambda i: (i, 0)),
            pl.BlockSpec((8, H), lambda i: (0, 0)),
            pl.BlockSpec((1, H), lambda i: (0, 0)),
            pl.BlockSpec((1, H), lambda i: (0, 0)),
            pl.BlockSpec((H, W), lambda i: (0, 0)),
            pl.BlockSpec((1, W), lambda i: (0, 0)),
            pl.BlockSpec((1, 1), lambda i: (0, 0), memory_space=pltpu.SMEM),
        ],
        out_specs=pl.BlockSpec((R, W), lambda i: (i, 0)),
        out_shape=jax.ShapeDtypeStruct((E, W), jnp.float32),
    )(h2, st2, g2, bt2, w3aug, b3aug, cnt)


def _sc_scatter(eaug, dst, N, K=80):
    E, W = eaug.shape
    epw = E // NW
    iters = epw // K
    Np = -(-N // (NS * 8)) * (NS * 8)
    rpw = Np // NS
    nv = W // 16
    mesh = plsc.VectorSubcoreMesh(core_axis_name="c", subcore_axis_name="s")

    @functools.partial(
        pl.kernel,
        out_type=jax.ShapeDtypeStruct((NC, Np, W), jnp.float32),
        mesh=mesh,
        scratch_types=[
            pltpu.VMEM((K,), jnp.int32),
            pltpu.VMEM((K, W), jnp.float32),
            pltpu.VMEM((rpw, W), jnp.float32),
            pltpu.VMEM((rpw, W), jnp.float32),
            pltpu.VMEM_SHARED((Np, W), jnp.float32),
            pltpu.SemaphoreType.DMA,
        ],
    )
    def scatter_k(eaug_hbm, dst_hbm, out_hbm, idxv, erows, zbuf, obuf,
                  shared, sem):
        cid = lax.axis_index("c")
        sid = lax.axis_index("s")
        wid = sid * NC + cid
        base0 = wid * epw
        r0 = sid * rpw

        def zrow(r, carry):
            for j in range(nv):
                zbuf[r, pl.ds(16 * j, 16)] = jnp.zeros((16,), jnp.float32)
            return carry

        lax.fori_loop(0, rpw, zrow, 0)
        pltpu.sync_copy(zbuf, shared.at[pl.ds(r0, rpw)])
        plsc.subcore_barrier()

        def body(i, carry):
            base = base0 + i * K
            pltpu.sync_copy(dst_hbm.at[pl.ds(base, K)], idxv)
            pltpu.sync_copy(eaug_hbm.at[pl.ds(base, K)], erows)
            pltpu.sync_copy(erows, shared.at[idxv], add=True)
            return carry

        lax.fori_loop(0, iters, body, 0)
        plsc.subcore_barrier()

        pltpu.sync_copy(shared.at[pl.ds(r0, rpw)], obuf)
        pltpu.sync_copy(obuf, out_hbm.at[cid, pl.ds(r0, rpw)])

    return scatter_k(eaug, dst)


def _node_body(part_ref, x_ref, w1m_ref, w1x_ref, b1_ref, g1_ref, bt1_ref,
               w2_ref, b2_ref, g2_ref, bt2_ref, w3_ref, b3_ref, out_ref):
    EO = w1m_ref.shape[0]
    Nn = x_ref.shape[0]
    p = part_ref[0, :Nn] + part_ref[1, :Nn]
    sums = p[:, :EO]
    cnt = p[:, EO:EO + 1]
    msg = sums / jnp.maximum(cnt, 1.0)
    x = x_ref[...]
    h = (jnp.dot(msg, w1m_ref[...], preferred_element_type=jnp.float32)
         + jnp.dot(x, w1x_ref[...], preferred_element_type=jnp.float32)
         + b1_ref[...])
    m = jnp.mean(h, axis=0, keepdims=True)
    v = jnp.mean(h * h, axis=0, keepdims=True) - m * m
    h = _silu(g1_ref[...] * (h - m) * lax.rsqrt(v + EPS) + bt1_ref[...])
    h = jnp.dot(h, w2_ref[...], preferred_element_type=jnp.float32) \
        + b2_ref[...]
    m = jnp.mean(h, axis=0, keepdims=True)
    v = jnp.mean(h * h, axis=0, keepdims=True) - m * m
    h = _silu(g2_ref[...] * (h - m) * lax.rsqrt(v + EPS) + bt2_ref[...])
    out_ref[...] = jnp.dot(h, w3_ref[...],
                           preferred_element_type=jnp.float32) \
        + b3_ref[...] + x


def _node(partials, x, w1m, w1x, b1, g1, bt1, w2, b2, g2, bt2, w3, b3):
    N, D = x.shape
    return pl.pallas_call(
        _node_body,
        out_shape=jax.ShapeDtypeStruct((N, D), jnp.float32),
    )(partials, x, w1m, w1x, b1, g1, bt1, w2, b2, g2, bt2, w3, b3)


def kernel(x, edge_index, edge_attr, eW1, eb1, eg1, ebt1, eW2, eb2, eg2,
           ebt2, eW3, eb3, nW1, nb1, ng1, nbt1, nW2, nb2, ng2, nbt2,
           nW3, nb3):
    N, D = x.shape
    E = edge_index.shape[1]
    DE = edge_attr.shape[1]
    H = eW2.shape[1]
    EO = eW3.shape[1]
    R = 4000

    src = edge_index[0]
    dst = edge_index[1]
    w1a = eW1[:D]
    w1b = eW1[D:2 * D]
    w1c = eW1[2 * D:]

    xa, xb = _prep(x, w1a, w1b)
    h1p = _sc_gather(xa, xb, dst, src)

    ecnt = jnp.full((1, 1), float(E), jnp.float32)
    st1 = _estats(h1p, edge_attr, w1c, eb1.reshape(1, -1), R)
    h2, st2 = _emain(h1p, edge_attr, w1c, eb1.reshape(1, -1), st1,
                     eg1.reshape(1, -1), ebt1.reshape(1, -1), eW2,
                     eb2.reshape(1, -1), ecnt, R)

    w3aug = jnp.concatenate([eW3, jnp.zeros((H, EO), jnp.float32)], axis=1)
    onecol = jnp.concatenate(
        [jnp.ones((1, 1), jnp.float32), jnp.zeros((1, EO - 1), jnp.float32)],
        axis=1)
    b3aug = jnp.concatenate([eb3.reshape(1, -1), onecol], axis=1)
    eaug = _efinal(h2, st2, eg2.reshape(1, -1), ebt2.reshape(1, -1),
                   w3aug, b3aug, ecnt, R)

    Np = -(-N // (NS * 8)) * (NS * 8)
    s = jax.ops.segment_sum(eaug, dst, num_segments=Np)
    partials = jnp.stack([s, jnp.zeros_like(s)])

    return _node(partials, x, nW1[:EO], nW1[EO:], nb1.reshape(1, -1),
                 ng1.reshape(1, -1), nbt1.reshape(1, -1), nW2,
                 nb2.reshape(1, -1), ng2.reshape(1, -1), nbt2.reshape(1, -1),
                 nW3, nb3.reshape(1, -1))
